# baseline (speedup 1.0000x reference)
import functools

import jax
import jax.numpy as jnp
from jax import lax
from jax.experimental import pallas as pl
from jax.experimental.pallas import tpu as pltpu
from jax.experimental.pallas import tpu_sc as plsc

EMB = 64
NC = 2
NS = 16
NW = NC * NS
LANES = 16
IDXW = 128
ROWW = EMB


def _vsqrt(x):
    i = lax.bitcast_convert_type(x, jnp.int32)
    y = lax.bitcast_convert_type((i >> 1) + jnp.int32(0x1FBD1DF6), jnp.float32)
    y = 0.5 * (y + x / y)
    y = 0.5 * (y + x / y)
    y = 0.5 * (y + x / y)
    return y


def _make_sc_kernel(tot, chunk, sub):
    ngather = sub // IDXW
    nsub = chunk // sub
    ngroups = sub // LANES
    nidx = chunk // IDXW

    mesh = plsc.VectorSubcoreMesh(core_axis_name="c", subcore_axis_name="s")

    @functools.partial(
        pl.kernel,
        out_type=jax.ShapeDtypeStruct((tot,), jnp.float32),
        mesh=mesh,
        scratch_types=dict(
            idx_h=pltpu.VMEM((nidx, IDXW), jnp.int32),
            idx_r=pltpu.VMEM((nidx, IDXW), jnp.int32),
            idx_t=pltpu.VMEM((nidx, IDXW), jnp.int32),
            rows_h=pltpu.VMEM((sub, ROWW), jnp.float32),
            rows_r=pltpu.VMEM((sub, ROWW), jnp.float32),
            rows_t=pltpu.VMEM((sub, ROWW), jnp.float32),
            out_v=pltpu.VMEM((chunk,), jnp.float32),
            sem=pltpu.SemaphoreType.DMA,
        ),
        compiler_params=pltpu.CompilerParams(
            needs_layout_passes=False, use_tc_tiling_on_sc=False),
    )
    def sc_kernel(heads_hbm, rels_hbm, tails_hbm, ent_hbm, rel_hbm, out_hbm,
                  *, idx_h, idx_r, idx_t, rows_h, rows_r, rows_t, out_v, sem):
        wid = lax.axis_index("s") * NC + lax.axis_index("c")
        base = pl.multiple_of(wid * chunk, chunk)
        lane = lax.iota(jnp.int32, 16)

        r0 = pl.multiple_of(base // IDXW, nidx)
        pltpu.sync_copy(heads_hbm.at[pl.ds(r0, nidx)], idx_h)
        pltpu.sync_copy(rels_hbm.at[pl.ds(r0, nidx)], idx_r)
        pltpu.sync_copy(tails_hbm.at[pl.ds(r0, nidx)], idx_t)

        for s in range(nsub):
            copies = []
            for j in range(ngather):
                src = s * ngather + j
                dst = pl.ds(j * IDXW, IDXW)
                copies.append(pltpu.async_copy(
                    ent_hbm.at[idx_h.at[src]], rows_h.at[dst], sem))
                copies.append(pltpu.async_copy(
                    rel_hbm.at[idx_r.at[src]], rows_r.at[dst], sem))
                copies.append(pltpu.async_copy(
                    ent_hbm.at[idx_t.at[src]], rows_t.at[dst], sem))
            for cp in copies:
                cp.wait()

            def group_body(g, _, s=s):
                vec = jnp.zeros((LANES,), jnp.float32)
                for k in range(LANES):
                    i = g * LANES + k
                    acc = jnp.zeros((LANES,), jnp.float32)
                    for q in range(EMB // LANES):
                        sl = pl.ds(q * LANES, LANES)
                        e = rows_h[i, sl] + rows_r[i, sl] - rows_t[i, sl]
                        acc = acc + e * e
                    vec = jnp.where(lane == k, jnp.sum(acc), vec)
                out_v[pl.ds(s * sub + g * LANES, LANES)] = _vsqrt(vec)
                return 0

            lax.fori_loop(0, sub // LANES, group_body, 0)

        pltpu.sync_copy(out_v, out_hbm.at[pl.ds(base, chunk)])

    return sc_kernel


def kernel(pos_triples, neg_triples, ent_embs, rel_embs):
    b = pos_triples.shape[0]
    tot = 2 * b
    chunk = tot // NW
    sub = min(chunk, 256)

    trip = jnp.concatenate(
        [pos_triples.astype(jnp.int32), neg_triples.astype(jnp.int32)], axis=0)
    heads = trip[:, 0].reshape(tot // IDXW, IDXW)
    rels = trip[:, 1].reshape(tot // IDXW, IDXW)
    tails = trip[:, 2].reshape(tot // IDXW, IDXW)

    out = _make_sc_kernel(tot, chunk, sub)(
        heads, rels, tails, ent_embs, rel_embs)
    return out[:b], out[b:]
